# Initial kernel scaffold; baseline (speedup 1.0000x reference)
import jax
import jax.numpy as jnp
from jax.experimental import pallas as pl


def kernel(edge_features, node_attrs, Wq1, Wq2, Wq3, Wk1, Wk2, Wk3, edge_index, node_types):
    raise NotImplementedError("write your pallas kernel here")


# speedup vs baseline: 1.0063x; 1.0063x over previous
import functools

import jax
import jax.numpy as jnp
import numpy as np
from jax.experimental import pallas as pl

N_SCALARS = 16
HEAD_DIM = 32
MUL = 16
FEAT_DIM = 64
ISQRTD = 5.0

BE = 1000


def _w_kernel(ga_ref, ef_ref, wq1_ref, wq2_ref, wq3_ref, wk1_ref, wk2_ref,
              wk3_ref, sel_ref, out_ref):
    ga = ga_ref[...]
    ef = ef_ref[...]
    q = jax.nn.silu(jnp.dot(ga, wq1_ref[...], preferred_element_type=jnp.float32))
    q = jax.nn.silu(jnp.dot(q, wq2_ref[...], preferred_element_type=jnp.float32))
    q = jnp.dot(q, wq3_ref[...], preferred_element_type=jnp.float32)
    k = jax.nn.silu(jnp.dot(ef, wk1_ref[...], preferred_element_type=jnp.float32))
    k = jax.nn.silu(jnp.dot(k, wk2_ref[...], preferred_element_type=jnp.float32))
    k = jnp.dot(k, wk3_ref[...], preferred_element_type=jnp.float32)
    qk = q * k
    out_ref[...] = jnp.dot(qk, sel_ref[...], preferred_element_type=jnp.float32) * ISQRTD


def _compute_w(ga, ef16, Wq1, Wq2, Wq3, Wk1, Wk2, Wk3):
    E = ga.shape[0]
    sel = jnp.repeat(jnp.eye(N_SCALARS, dtype=jnp.float32), HEAD_DIM, axis=0)
    grid = (E // BE,)
    return pl.pallas_call(
        _w_kernel,
        grid=grid,
        in_specs=[
            pl.BlockSpec((BE, 64), lambda i: (i, 0)),
            pl.BlockSpec((BE, N_SCALARS), lambda i: (i, 0)),
            pl.BlockSpec((64, 64), lambda i: (0, 0)),
            pl.BlockSpec((64, 64), lambda i: (0, 0)),
            pl.BlockSpec((64, 512), lambda i: (0, 0)),
            pl.BlockSpec((N_SCALARS, 64), lambda i: (0, 0)),
            pl.BlockSpec((64, 64), lambda i: (0, 0)),
            pl.BlockSpec((64, 512), lambda i: (0, 0)),
            pl.BlockSpec((512, N_SCALARS), lambda i: (0, 0)),
        ],
        out_specs=pl.BlockSpec((BE, N_SCALARS), lambda i: (i, 0)),
        out_shape=jax.ShapeDtypeStruct((E, N_SCALARS), jnp.float32),
    )(ga, ef16, Wq1, Wq2, Wq3, Wk1, Wk2, Wk3, sel)


def kernel(edge_features, node_attrs, Wq1, Wq2, Wq3, Wk1, Wk2, Wk3, edge_index, node_types):
    E = edge_features.shape[0]
    num_nodes = node_types.shape[0]
    edge_center = edge_index[0]

    ga = node_attrs[edge_center]
    W = _compute_w(ga, edge_features[:, :N_SCALARS], Wq1, Wq2, Wq3, Wk1, Wk2, Wk3)

    m = jax.ops.segment_max(W, edge_center, num_segments=num_nodes)
    m = jnp.where(jnp.isfinite(m), m, 0.0)
    w_exp = jnp.exp(W - m[edge_center])
    denom = jax.ops.segment_sum(w_exp, edge_center, num_segments=num_nodes)
    alpha = w_exp / denom[edge_center]
    scal = edge_features[:, :MUL].reshape(E, MUL, 1)
    vec = edge_features[:, MUL:].reshape(E, MUL, 3)
    ef = jnp.concatenate([scal, vec], axis=-1)
    ef = ef * alpha[:, :, None]
    out_edge = jnp.concatenate([ef[:, :, 0], ef[:, :, 1:].reshape(E, MUL * 3)], axis=-1)
    return jax.ops.segment_sum(out_edge, edge_center, num_segments=num_nodes)


# speedup vs baseline: 2.0703x; 2.0573x over previous
import functools

import jax
import jax.numpy as jnp
import numpy as np
from jax import lax
from jax.experimental import pallas as pl
from jax.experimental.pallas import tpu as pltpu
from jax.experimental.pallas import tpu_sc as plsc

_SC_MESH = plsc.VectorSubcoreMesh(core_axis_name="c", subcore_axis_name="s")
NW = 32

N_SCALARS = 16
HEAD_DIM = 32
MUL = 16
FEAT_DIM = 64
ISQRTD = 5.0

BE = 1000


def _w_kernel(ga_ref, ef_ref, wq1_ref, wq2_ref, wq3_ref, wk1_ref, wk2_ref,
              wk3_ref, sel_ref, out_ref, out128_ref):
    ga = ga_ref[:, :64]
    ef = ef_ref[...]
    q = jax.nn.silu(jnp.dot(ga, wq1_ref[...], preferred_element_type=jnp.float32))
    q = jax.nn.silu(jnp.dot(q, wq2_ref[...], preferred_element_type=jnp.float32))
    q = jnp.dot(q, wq3_ref[...], preferred_element_type=jnp.float32)
    k = jax.nn.silu(jnp.dot(ef, wk1_ref[...], preferred_element_type=jnp.float32))
    k = jax.nn.silu(jnp.dot(k, wk2_ref[...], preferred_element_type=jnp.float32))
    k = jnp.dot(k, wk3_ref[...], preferred_element_type=jnp.float32)
    qk = q * k
    w128 = jnp.dot(qk, sel_ref[...], preferred_element_type=jnp.float32) * ISQRTD
    out128_ref[...] = w128
    out_ref[...] = w128[:, :N_SCALARS]


def _compute_w(ga, ef16, Wq1, Wq2, Wq3, Wk1, Wk2, Wk3):
    E = ga.shape[0]
    sel = jnp.repeat(jnp.eye(N_SCALARS, dtype=jnp.float32), HEAD_DIM, axis=0)
    sel = jnp.pad(sel, ((0, 0), (0, 128 - N_SCALARS)))
    grid = (E // BE,)
    return pl.pallas_call(
        _w_kernel,
        grid=grid,
        in_specs=[
            pl.BlockSpec((BE, 128), lambda i: (i, 0)),
            pl.BlockSpec((BE, N_SCALARS), lambda i: (i, 0)),
            pl.BlockSpec((64, 64), lambda i: (0, 0)),
            pl.BlockSpec((64, 64), lambda i: (0, 0)),
            pl.BlockSpec((64, 512), lambda i: (0, 0)),
            pl.BlockSpec((N_SCALARS, 64), lambda i: (0, 0)),
            pl.BlockSpec((64, 64), lambda i: (0, 0)),
            pl.BlockSpec((64, 512), lambda i: (0, 0)),
            pl.BlockSpec((512, 128), lambda i: (0, 0)),
        ],
        out_specs=[
            pl.BlockSpec((BE, N_SCALARS), lambda i: (i, 0)),
            pl.BlockSpec((BE, 128), lambda i: (i, 0)),
        ],
        out_shape=[
            jax.ShapeDtypeStruct((E, N_SCALARS), jnp.float32),
            jax.ShapeDtypeStruct((E, 128), jnp.float32),
        ],
    )(ga, ef16, Wq1, Wq2, Wq3, Wk1, Wk2, Wk3, sel)


def _gather_body(centers2_hbm, table_hbm, out_hbm, idx_v, rows_v, sem):
    n_chunks = centers2_hbm.shape[0]
    w = lax.axis_index("s") * 2 + lax.axis_index("c")

    def body(t, carry):
        cidx = w + NW * t

        @pl.when(cidx < n_chunks)
        def _():
            pltpu.sync_copy(centers2_hbm.at[cidx], idx_v)
            pltpu.async_copy(table_hbm.at[idx_v], rows_v, sem).wait()
            pltpu.sync_copy(rows_v, out_hbm.at[pl.ds(cidx * 64, 64)])

        return carry

    lax.fori_loop(0, (n_chunks + NW - 1) // NW, body, 0)


def _sc_gather(centers, table):
    E = centers.shape[0]
    D = table.shape[1]
    centers2 = centers.reshape(E // 64, 64)
    return pl.kernel(
        _gather_body,
        out_type=jax.ShapeDtypeStruct((E, D), jnp.float32),
        mesh=_SC_MESH,
        scratch_types=[
            pltpu.VMEM((64,), jnp.int32),
            pltpu.VMEM((64, D), jnp.float32),
            pltpu.SemaphoreType.DMA,
        ],
    )(centers2, table)


NPAD = 10240
NODES_PER_W = 320
SCAN_CH = 3200
N_SCAN_CH = 50
GB = 64


def _seg_max_body(centers_hbm, w128_hbm, m_hbm, scanbuf, cid, cct, idx_g,
                  rows_g, macc, cbr, sem):
    w = lax.axis_index("s") * 2 + lax.axis_index("c")
    base = w * NODES_PER_W
    iota = lax.iota(jnp.int32, 16)
    neginf = jnp.full((16,), -jnp.inf, dtype=jnp.float32)
    zeros_i = jnp.zeros((16,), dtype=jnp.int32)

    def init_macc(i, c):
        macc[i, pl.ds(0, 16)] = neginf
        return c

    lax.fori_loop(0, NODES_PER_W, init_macc, 0)

    def init_cbuf(i, c):
        cid[pl.ds(i * 16, 16)] = zeros_i
        cct[pl.ds(i * 16, 16)] = zeros_i
        return c

    lax.fori_loop(0, SCAN_CH // 16, init_cbuf, 0)

    def chunk_body(ch, carry):
        pltpu.sync_copy(centers_hbm.at[pl.ds(ch * SCAN_CH, SCAN_CH)], scanbuf)

        def scan_group(g, cnt):
            v = scanbuf[pl.ds(g * 16, 16)]
            rel = v - base
            msk = (rel >= 0) & (rel < NODES_PER_W)
            ids = jnp.full((16,), ch * SCAN_CH + g * 16, jnp.int32) + iota
            plsc.store_compressed(cid.at[pl.ds(cnt, 16)], ids, mask=msk)
            plsc.store_compressed(cct.at[pl.ds(cnt, 16)], rel, mask=msk)
            cbr[pl.ds(0, 16)] = plsc.all_reduce_population_count(msk)
            return cnt + cbr[pl.ds(0, 16)][0]

        cnt = lax.fori_loop(0, SCAN_CH // 16, scan_group, 0)

        def cand_batch(gb, carry2):
            pltpu.async_copy(w128_hbm.at[cid.at[pl.ds(gb * GB, GB)]], rows_g,
                             sem).wait()

            def sub_group(sg, carry3):
                lnv = cct[pl.ds(gb * GB + sg * 16, 16)]
                for j in range(16):
                    pos = gb * GB + sg * 16 + j

                    @pl.when(pos < cnt)
                    def _():
                        ln = lnv[j]
                        wrow = rows_g[sg * 16 + j, pl.ds(0, 16)]
                        cur = macc[ln, pl.ds(0, 16)]
                        macc[ln, pl.ds(0, 16)] = jnp.maximum(cur, wrow)

                return carry3

            lax.fori_loop(0, GB // 16, sub_group, 0)
            return carry2

        lax.fori_loop(0, (cnt + GB - 1) // GB, cand_batch, 0)
        return carry

    lax.fori_loop(0, N_SCAN_CH, chunk_body, 0)
    pltpu.sync_copy(macc, m_hbm.at[pl.ds(base, NODES_PER_W)])


def _seg_max(centers, w128):
    return pl.kernel(
        _seg_max_body,
        out_type=jax.ShapeDtypeStruct((NPAD, 16), jnp.float32),
        mesh=_SC_MESH,
        compiler_params=pltpu.CompilerParams(needs_layout_passes=False),
        scratch_types=[
            pltpu.VMEM((SCAN_CH,), jnp.int32),
            pltpu.VMEM((SCAN_CH,), jnp.int32),
            pltpu.VMEM((SCAN_CH,), jnp.int32),
            pltpu.VMEM((GB,), jnp.int32),
            pltpu.VMEM((GB, 128), jnp.float32),
            pltpu.VMEM((NODES_PER_W, 16), jnp.float32),
            pltpu.VMEM((16,), jnp.int32),
            pltpu.SemaphoreType.DMA,
        ],
    )(centers, w128)


def _denom_body(centers_hbm, w16_hbm, m_hbm, dacc_hbm, wexp_hbm,
                m_sp, dacc_sp, ctrv, wbuf, mbuf, wxbuf, sem):
    c = lax.axis_index("c")
    s = lax.axis_index("s")
    w = s * 2 + c
    zeros = jnp.zeros((16,), dtype=jnp.float32)
    rows_per_s = NPAD // 16

    def zinit(i, carry):
        wxbuf[i, pl.ds(0, 16)] = zeros
        return carry

    lax.fori_loop(0, 128, zinit, 0)
    pltpu.sync_copy(m_hbm.at[pl.ds(s * rows_per_s, rows_per_s)],
                    m_sp.at[pl.ds(s * rows_per_s, rows_per_s)])
    for k in range(rows_per_s // 128):
        pltpu.sync_copy(wxbuf, dacc_sp.at[pl.ds(s * rows_per_s + k * 128, 128)])
    plsc.subcore_barrier()

    n_chunks = 160000 // 128

    def chunk_body(t, carry):
        cidx = w + NW * t

        @pl.when(cidx < n_chunks)
        def _():
            e0 = cidx * 128
            pltpu.sync_copy(centers_hbm.at[pl.ds(e0, 128)], ctrv)
            pltpu.sync_copy(w16_hbm.at[pl.ds(e0, 128)], wbuf)
            pltpu.async_copy(m_sp.at[ctrv], mbuf, sem).wait()

            def edge_body(e, carry2):
                wrow = wbuf[e, pl.ds(0, 16)]
                mrow = mbuf[e, pl.ds(0, 16)]
                wxbuf[e, pl.ds(0, 16)] = jnp.exp(wrow - mrow)
                return carry2

            lax.fori_loop(0, 128, edge_body, 0)
            pltpu.sync_copy(wxbuf, wexp_hbm.at[pl.ds(e0, 128)])
            pltpu.sync_copy(wxbuf, dacc_sp.at[ctrv], add=True)

        return carry

    lax.fori_loop(0, (n_chunks + NW - 1) // NW, chunk_body, 0)
    plsc.subcore_barrier()

    pltpu.sync_copy(dacc_sp.at[pl.ds(s * rows_per_s, rows_per_s)],
                    dacc_hbm.at[c, pl.ds(s * rows_per_s, rows_per_s)])


def _denom(centers, w16, m):
    return pl.kernel(
        _denom_body,
        out_type=[
            jax.ShapeDtypeStruct((2, NPAD, 16), jnp.float32),
            jax.ShapeDtypeStruct((160000, 16), jnp.float32),
        ],
        mesh=_SC_MESH,
        compiler_params=pltpu.CompilerParams(needs_layout_passes=False),
        scratch_types=[
            pltpu.VMEM_SHARED((NPAD, 16), jnp.float32),
            pltpu.VMEM_SHARED((NPAD, 16), jnp.float32),
            pltpu.VMEM((128,), jnp.int32),
            pltpu.VMEM((128, 16), jnp.float32),
            pltpu.VMEM((128, 16), jnp.float32),
            pltpu.VMEM((128, 16), jnp.float32),
            pltpu.SemaphoreType.DMA,
        ],
    )(centers, w16, m)


def _wsum_body(centers_hbm, ef_hbm, wexp_hbm, oacc_hbm,
               oacc_sp, ctrv, efbuf, wxbuf, obuf, sem):
    c = lax.axis_index("c")
    s = lax.axis_index("s")
    w = s * 2 + c
    iota = lax.iota(jnp.int32, 16)
    zeros = jnp.zeros((16,), dtype=jnp.float32)
    p0 = iota // 3
    p1 = (iota + 16) // 3
    p2 = (iota + 32) // 3
    rows_per_s = NPAD // 16

    def zinit(i, carry):
        for k in range(4):
            obuf[i, pl.ds(k * 16, 16)] = zeros
        return carry

    lax.fori_loop(0, 128, zinit, 0)
    for k in range(rows_per_s // 128):
        pltpu.sync_copy(obuf, oacc_sp.at[pl.ds(s * rows_per_s + k * 128, 128)])
    plsc.subcore_barrier()

    n_chunks = 160000 // 128

    def chunk_body(t, carry):
        cidx = w + NW * t

        @pl.when(cidx < n_chunks)
        def _():
            e0 = cidx * 128
            pltpu.sync_copy(centers_hbm.at[pl.ds(e0, 128)], ctrv)
            pltpu.sync_copy(ef_hbm.at[pl.ds(e0, 128)], efbuf)
            pltpu.sync_copy(wexp_hbm.at[pl.ds(e0, 128)], wxbuf)

            def edge_body(e, carry2):
                wx = wxbuf[e, pl.ds(0, 16)]
                a0 = wx.at[p0].get(mode="promise_in_bounds")
                a1 = wx.at[p1].get(mode="promise_in_bounds")
                a2 = wx.at[p2].get(mode="promise_in_bounds")
                obuf[e, pl.ds(0, 16)] = efbuf[e, pl.ds(0, 16)] * wx
                obuf[e, pl.ds(16, 16)] = efbuf[e, pl.ds(16, 16)] * a0
                obuf[e, pl.ds(32, 16)] = efbuf[e, pl.ds(32, 16)] * a1
                obuf[e, pl.ds(48, 16)] = efbuf[e, pl.ds(48, 16)] * a2
                return carry2

            lax.fori_loop(0, 128, edge_body, 0)
            pltpu.sync_copy(obuf, oacc_sp.at[ctrv], add=True)

        return carry

    lax.fori_loop(0, (n_chunks + NW - 1) // NW, chunk_body, 0)
    plsc.subcore_barrier()

    pltpu.sync_copy(oacc_sp.at[pl.ds(s * rows_per_s, rows_per_s)],
                    oacc_hbm.at[c, pl.ds(s * rows_per_s, rows_per_s)])


def _wsum(centers, ef, wexp):
    return pl.kernel(
        _wsum_body,
        out_type=jax.ShapeDtypeStruct((2, NPAD, 64), jnp.float32),
        mesh=_SC_MESH,
        compiler_params=pltpu.CompilerParams(needs_layout_passes=False),
        scratch_types=[
            pltpu.VMEM_SHARED((NPAD, 64), jnp.float32),
            pltpu.VMEM((128,), jnp.int32),
            pltpu.VMEM((128, 64), jnp.float32),
            pltpu.VMEM((128, 16), jnp.float32),
            pltpu.VMEM((128, 64), jnp.float32),
            pltpu.SemaphoreType.DMA,
        ],
    )(centers, ef, wexp)


def _final_kernel(dacc_ref, oacc_ref, p_ref, out_ref):
    d = dacc_ref[0] + dacc_ref[1]
    o = oacc_ref[0] + oacc_ref[1]
    r = jnp.where(d > 0.0, 1.0 / d, 0.0)
    out_ref[...] = o * jnp.dot(r, p_ref[...], preferred_element_type=jnp.float32)


def _finalize(dacc, oacc, num_nodes):
    pmat = np.zeros((16, 64), dtype=np.float32)
    for col in range(64):
        pmat[col if col < 16 else (col - 16) // 3, col] = 1.0
    BN = 1024
    out = pl.pallas_call(
        _final_kernel,
        grid=(NPAD // BN,),
        in_specs=[
            pl.BlockSpec((2, BN, 16), lambda i: (0, i, 0)),
            pl.BlockSpec((2, BN, 64), lambda i: (0, i, 0)),
            pl.BlockSpec((16, 64), lambda i: (0, 0)),
        ],
        out_specs=pl.BlockSpec((BN, 64), lambda i: (i, 0)),
        out_shape=jax.ShapeDtypeStruct((NPAD, 64), jnp.float32),
    )(dacc, oacc, jnp.asarray(pmat))
    return out[:num_nodes]


def kernel(edge_features, node_attrs, Wq1, Wq2, Wq3, Wk1, Wk2, Wk3, edge_index, node_types):
    E = edge_features.shape[0]
    num_nodes = node_types.shape[0]
    edge_center = edge_index[0]

    na_pad = jnp.pad(node_attrs, ((0, 0), (0, 128 - node_attrs.shape[1])))
    ga = _sc_gather(edge_center, na_pad)
    w16, w128 = _compute_w(ga, edge_features[:, :N_SCALARS],
                           Wq1, Wq2, Wq3, Wk1, Wk2, Wk3)
    m = _seg_max(edge_center, w128)
    dacc, wexp = _denom(edge_center, w16, m)
    oacc = _wsum(edge_center, edge_features, wexp)
    return _finalize(dacc, oacc, num_nodes)
